# Initial kernel scaffold; baseline (speedup 1.0000x reference)
import jax
import jax.numpy as jnp
from jax.experimental import pallas as pl


def kernel(feat_node, feat_edge, feat_face, ei_nn, ei_ee, ei_ff, ei_ne, ei_ef, params):
    raise NotImplementedError("write your pallas kernel here")


# speedup vs baseline: 2.9010x; 2.9010x over previous
import functools
import jax
import jax.numpy as jnp
from jax.experimental import pallas as pl

_BN = 1000
_CN = 1000
_BE = 640


def _linear_body(x_ref, w_ref, b_ref, o_ref):
    y = jnp.dot(x_ref[...], w_ref[...], preferred_element_type=jnp.float32)
    o_ref[...] = jnp.maximum(y + b_ref[0:1, :], 0.0)


def _linear(x, w, b):
    n = x.shape[0]
    b8 = jnp.broadcast_to(b[None, :], (8, w.shape[1]))
    return pl.pallas_call(
        _linear_body,
        grid=(n // _BN,),
        in_specs=[
            pl.BlockSpec((_BN, x.shape[1]), lambda i: (i, 0)),
            pl.BlockSpec(w.shape, lambda i: (0, 0)),
            pl.BlockSpec((8, w.shape[1]), lambda i: (0, 0)),
        ],
        out_specs=pl.BlockSpec((_BN, w.shape[1]), lambda i: (i, 0)),
        out_shape=jax.ShapeDtypeStruct((n, w.shape[1]), jnp.float32),
    )(x, w, b8)


def _proj_body(x_ref, w_ref, al_ref, ar_ref, t_ref, er_ref):
    fs = jnp.dot(x_ref[...], w_ref[...], preferred_element_type=jnp.float32)
    el = jnp.dot(fs, al_ref[...], preferred_element_type=jnp.float32)
    er = jnp.dot(fs, ar_ref[...], preferred_element_type=jnp.float32)
    t_ref[...] = jnp.concatenate([fs, el], axis=1)
    er_ref[...] = er


def _proj(x, w, al_pad, ar_pad):
    n = x.shape[0]
    return pl.pallas_call(
        _proj_body,
        grid=(n // _BN,),
        in_specs=[
            pl.BlockSpec((_BN, x.shape[1]), lambda i: (i, 0)),
            pl.BlockSpec(w.shape, lambda i: (0, 0)),
            pl.BlockSpec((512, 128), lambda i: (0, 0)),
            pl.BlockSpec((512, 128), lambda i: (0, 0)),
        ],
        out_specs=[
            pl.BlockSpec((_BN, 640), lambda i: (i, 0)),
            pl.BlockSpec((_BN, 128), lambda i: (i, 0)),
        ],
        out_shape=[
            jax.ShapeDtypeStruct((n, 640), jnp.float32),
            jax.ShapeDtypeStruct((n, 128), jnp.float32),
        ],
    )(x, w, al_pad, ar_pad)


def _edge_body(t_ref, srcc_ref, dstc_ref, dstr_ref, r8_ref, o_ref, *, nc):
    i = pl.program_id(0)
    srcc = srcc_ref[0]
    dstc = dstc_ref[0]
    dstr = dstr_ref[0]
    gs = jnp.zeros((_BE, 640), jnp.float32)
    erd = jnp.zeros((_BE, 128), jnp.float32)
    for c in range(nc):
        ids = jax.lax.broadcasted_iota(jnp.int32, (_BE, _CN), 1) + c * _CN
        ohs = (srcc == ids).astype(jnp.float32)
        ohd = (dstc == ids).astype(jnp.float32)
        gs = gs + jnp.dot(ohs, t_ref[c * _CN:(c + 1) * _CN, :],
                          preferred_element_type=jnp.float32)
        erd = erd + jnp.dot(ohd, t_ref[c * _CN:(c + 1) * _CN, 512:640],
                            preferred_element_type=jnp.float32)
    e = gs[:, 512:520] + erd[:, 8:16]
    e = jnp.where(e >= 0.0, e, 0.2 * e)
    ex = jnp.exp(e)
    exw = jnp.dot(ex, r8_ref[...], preferred_element_type=jnp.float32)
    exfs = gs[:, 0:512] * exw
    vals = jnp.concatenate(
        [exfs, ex, jnp.zeros((_BE, 120), jnp.float32)], axis=1)
    for c in range(nc):
        idc = jax.lax.broadcasted_iota(jnp.int32, (_CN, _BE), 0) + c * _CN
        oht = (idc == dstr).astype(jnp.float32)
        acc = jnp.dot(oht, vals, preferred_element_type=jnp.float32)
        prev = o_ref[c * _CN:(c + 1) * _CN, :]
        o_ref[c * _CN:(c + 1) * _CN, :] = jnp.where(i == 0, acc, prev + acc)


def _edge(t, ei, r8):
    n = t.shape[0]
    e = ei.shape[1]
    eb = e // _BE
    srcc = ei[0].reshape(eb, _BE, 1)
    dstc = ei[1].reshape(eb, _BE, 1)
    dstr = ei[1].reshape(eb, 1, _BE)
    body = functools.partial(_edge_body, nc=n // _CN)
    return pl.pallas_call(
        body,
        grid=(eb,),
        in_specs=[
            pl.BlockSpec((n, 640), lambda i: (0, 0)),
            pl.BlockSpec((1, _BE, 1), lambda i: (i, 0, 0)),
            pl.BlockSpec((1, _BE, 1), lambda i: (i, 0, 0)),
            pl.BlockSpec((1, 1, _BE), lambda i: (i, 0, 0)),
            pl.BlockSpec((8, 512), lambda i: (0, 0)),
        ],
        out_specs=pl.BlockSpec((n, 640), lambda i: (0, 0)),
        out_shape=jax.ShapeDtypeStruct((n, 640), jnp.float32),
    )(t, srcc, dstc, dstr, r8)


def _combine_body(*refs, k):
    q = refs[0][...]
    r8 = refs[k + 3][...]
    zs = [q]
    for t in range(1, k):
        o = refs[t][...]
        den = jnp.dot(o[:, 512:520], r8, preferred_element_type=jnp.float32)
        gat = o[:, 0:512] / jnp.maximum(den, 1e-9)
        zs.append(jnp.where(gat > 0.0, gat,
                            jnp.exp(jnp.minimum(gat, 0.0)) - 1.0))
    a1 = refs[k][...]
    b1 = refs[k + 1][0:1, :]
    a2 = refs[k + 2][...]
    ys = [jnp.dot(jnp.tanh(jnp.dot(z, a1, preferred_element_type=jnp.float32)
                           + b1), a2,
                  preferred_element_type=jnp.float32)[:, 0:1] for z in zs]
    m = ys[0]
    for y in ys[1:]:
        m = jnp.maximum(m, y)
    es = [jnp.exp(y - m) for y in ys]
    tot = es[0]
    for e2 in es[1:]:
        tot = tot + e2
    s = zs[0] * (es[0] / tot)
    for t in range(1, k):
        s = s + zs[t] * (es[t] / tot)
    mu = jnp.mean(s, axis=1, keepdims=True)
    v = jnp.mean((s - mu) * (s - mu), axis=1, keepdims=True)
    refs[-1][...] = ((s - mu) * jax.lax.rsqrt(v + 1e-5)
                     * refs[k + 4][0:1, :] + refs[k + 5][0:1, :])


def _combine(q, os_, a1, b1, a2, r8, g, b):
    k = 1 + len(os_)
    n = q.shape[0]
    b1p = jnp.broadcast_to(b1[None, :], (8, 128))
    a2p = jnp.concatenate([a2, jnp.zeros((128, 127), jnp.float32)], axis=1)
    g8 = jnp.broadcast_to(g[None, :], (8, 512))
    bb8 = jnp.broadcast_to(b[None, :], (8, 512))
    body = functools.partial(_combine_body, k=k)
    in_specs = [pl.BlockSpec((_BN, 512), lambda i: (i, 0))]
    in_specs += [pl.BlockSpec((_BN, 640), lambda i: (i, 0)) for _ in os_]
    in_specs += [pl.BlockSpec(s, lambda i: (0, 0))
                 for s in [(512, 128), (8, 128), (128, 128),
                           (8, 512), (8, 512), (8, 512)]]
    return pl.pallas_call(
        body,
        grid=(n // _BN,),
        in_specs=in_specs,
        out_specs=pl.BlockSpec((_BN, 512), lambda i: (i, 0)),
        out_shape=jax.ShapeDtypeStruct((n, 512), jnp.float32),
    )(q, *os_, a1, b1p, a2p, r8, g8, bb8)


def _head_mat(al):
    m = jnp.zeros((512, 128), jnp.float32)
    return m.at[jnp.arange(512), jnp.arange(512) // 64].set(al.reshape(-1))


def kernel(feat_node, feat_edge, feat_face, ei_nn, ei_ee, ei_ff, ei_ne, ei_ef, params):
    p = params
    r8 = ((jnp.arange(512)[None, :] // 64)
          == jnp.arange(8)[:, None]).astype(jnp.float32)

    qn = _linear(feat_node, p['Wq_node'], p['bq_node'])
    qe = _linear(feat_edge, p['Wq_edge'], p['bq_edge'])
    qf = _linear(feat_face, p['Wq_face'], p['bq_face'])

    n = feat_node.shape[0]
    zpad = jnp.zeros((n, 112), jnp.float32)

    def table(t, er):
        return jnp.concatenate([t[:, 0:520], er[:, 0:8], zpad], axis=1)

    outs = {}
    for nm, x, ei in [('nn', feat_node, ei_nn), ('ee', feat_edge, ei_ee),
                      ('ff', feat_face, ei_ff)]:
        t, er = _proj(x, p['W_' + nm], _head_mat(p['al_' + nm]),
                      _head_mat(p['ar_' + nm]))
        outs[nm] = _edge(table(t, er), ei.astype(jnp.int32), r8)
    for nm, xs, xd, ei in [('ne', feat_node, feat_edge, ei_ne),
                           ('ef', feat_edge, feat_face, ei_ef)]:
        t, _ = _proj(xs, p['Ws_' + nm], _head_mat(p['al_' + nm]),
                     _head_mat(p['ar_' + nm]))
        _, er = _proj(xd, p['Wd_' + nm], _head_mat(p['al_' + nm]),
                      _head_mat(p['ar_' + nm]))
        outs[nm] = _edge(table(t, er), ei.astype(jnp.int32), r8)

    on = _combine(qn, [outs['nn']], p['A1_node'], p['a1b_node'],
                  p['A2_node'], r8, p['g_node'], p['b_node'])
    oe = _combine(qe, [outs['ee'], outs['ne']], p['A1_edge'], p['a1b_edge'],
                  p['A2_edge'], r8, p['g_edge'], p['b_edge'])
    of_ = _combine(qf, [outs['ff'], outs['ef']], p['A1_face'], p['a1b_face'],
                   p['A2_face'], r8, p['g_face'], p['b_face'])
    return (on, oe, of_)
